# baseline (speedup 1.0000x reference)
import functools

import jax
import jax.numpy as jnp
import numpy as np
from jax import lax
from jax.experimental import pallas as pl
from jax.experimental.pallas import tpu as pltpu
from jax.experimental.pallas import tpu_sc as plsc

_LEVELS = np.array([8, 5, 5, 5])
_RAW_DIMS = np.array([7, 9, 10, 11, 12])
_B = 16384
_R = 1024
_NT = 5
_NC = 2
_NS = 16
_NW = _NC * _NS
_BPW = _B // _NW
_TD = 16

_EPS = 1e-3
_HALF_L = ((_LEVELS - 1.0) * (1.0 + _EPS) / 2.0).astype(np.float32)
_OFFSET = np.where(_LEVELS % 2 == 0, 0.5, 0.0).astype(np.float32)
_SHIFT = np.arctanh(_OFFSET / _HALF_L).astype(np.float32)
_HALF_W = (_LEVELS // 2).astype(np.float32)
_BASIS = np.concatenate([[1], np.cumprod(_LEVELS[:-1])]).astype(np.float32)
_MASK_TABLE = (np.arange(12)[None, :] < _RAW_DIMS[:, None]).astype(np.float32)


def _sc_body(st_hbm, out_hbm, st_v, cols_v):
    wid = lax.axis_index("s") * _NC + lax.axis_index("c")
    base = wid * _BPW
    pltpu.sync_copy(st_hbm.at[pl.ds(base, _BPW)], st_v)

    def body(g):
        st16 = st_v[pl.ds(g * 16, 16)]
        rd16 = jnp.full((16,), int(_RAW_DIMS[0]), jnp.int32)
        for t in range(1, _NT):
            d = st16 - t
            eq = 1 - jnp.minimum(d * d, 1)
            rd16 = rd16 + eq * int(_RAW_DIMS[t] - _RAW_DIMS[0])
        for c in range(12):
            col = jnp.minimum(jnp.maximum(rd16 - c, 0), 1)
            cols_v[c, pl.ds(g * 16, 16)] = col.astype(jnp.float32)

    for g in range(_BPW // 16):
        body(g)
    for c in range(12):
        pltpu.sync_copy(cols_v.at[c], out_hbm.at[c, pl.ds(base, _BPW)])


_sc_route_mask = functools.partial(
    pl.kernel,
    out_type=jax.ShapeDtypeStruct((12, _B), jnp.float32),
    mesh=plsc.VectorSubcoreMesh(core_axis_name="c", subcore_axis_name="s"),
    scratch_types=[
        pltpu.VMEM((_BPW,), jnp.int32),
        pltpu.VMEM((12, _BPW), jnp.float32),
    ],
)(_sc_body)


def _tc_body(stf_ref, params_ref,
             wpeT_ref, bpe_ref,
             w1aT_ref, w1bT_ref, b1_ref,
             w2T_ref, b2_ref, w3T_ref, b3_ref, w4T_ref, b4_ref,
             fwinT_ref, fbin_ref, fwoutT_ref, fbout_ref,
             clsT_ref, clsb_ref, iscT_ref, iscb_ref,
             d1aT_ref, d1bT_ref, db1_ref, d2T_ref, db2_ref, d3T_ref, db3_ref,
             wdrT_ref, bdr_ref, temb_ref,
             shift_ref, halfl_ref, offs_ref, halfw_ref, basis_ref,
             recon_ref, cls_ref, isc_ref, zq_ref, idx_ref):
    f32 = jnp.float32
    dot = functools.partial(jnp.dot, preferred_element_type=f32)
    sti = stf_ref[...]
    iota5 = jax.lax.broadcasted_iota(jnp.int32, (_R, _NT), 1)
    onehot = (iota5 == sti).astype(f32)
    emb = dot(onehot, temb_ref[...])

    p5 = dot(params_ref[...], wpeT_ref[...]) + bpe_ref[...]
    pe = onehot[:, 0:1] * p5[:, 0:32]
    for t in range(1, _NT):
        pe = pe + onehot[:, t:t + 1] * p5[:, 32 * t:32 * (t + 1)]

    bf = jnp.bfloat16
    h = jnp.maximum(dot(pe.astype(bf), w1aT_ref[...]) + dot(emb.astype(bf), w1bT_ref[...]) + b1_ref[...], 0.0)
    h = jnp.maximum(dot(h.astype(bf), w2T_ref[...]) + b2_ref[...], 0.0)
    h = jnp.maximum(dot(h.astype(bf), w3T_ref[...]) + b3_ref[...], 0.0)
    z = dot(h.astype(bf), w4T_ref[...]) + b4_ref[...]

    zp = dot(z, fwinT_ref[...]) + fbin_ref[...]
    bounded = jnp.tanh(zp + shift_ref[...]) * halfl_ref[...] - offs_ref[...]
    rounded = jnp.round(bounded)
    codes = rounded / halfw_ref[...]
    idx_f = jnp.sum((rounded + halfw_ref[...]) * basis_ref[...],
                    axis=1, keepdims=True)
    idx_ref[...] = idx_f.astype(jnp.int32)
    zq = dot(codes, fwoutT_ref[...]) + fbout_ref[...]
    zq_ref[...] = zq

    cls_ref[...] = dot(zq, clsT_ref[...]) + clsb_ref[...]
    isc_ref[...] = dot(zq, iscT_ref[...]) + iscb_ref[...]

    hd = jnp.maximum(dot(zq.astype(bf), d1aT_ref[...]) + dot(emb.astype(bf), d1bT_ref[...]) + db1_ref[...], 0.0)
    hd = jnp.maximum(dot(hd.astype(bf), d2T_ref[...]) + db2_ref[...], 0.0)
    pd = dot(hd.astype(bf), d3T_ref[...]) + db3_ref[...]

    d5 = dot(pd, wdrT_ref[...]) + bdr_ref[...]
    recon = onehot[:, 0:1] * d5[:, 0:12]
    for t in range(1, _NT):
        recon = recon + onehot[:, t:t + 1] * d5[:, 12 * t:12 * (t + 1)]
    recon_ref[...] = recon


def _full(shape):
    nd = len(shape)
    return pl.BlockSpec(shape, lambda i: (0,) * nd)


def _rows(width):
    return pl.BlockSpec((_R, width), lambda i: (i, 0))


@jax.jit
def _run(stf, params, args):
    maskf = _sc_route_mask(stf.reshape(_B))
    grid = _B // _R
    in_specs = [_rows(1), _rows(12)] + [_full(a.shape) for a in args]
    out_shapes = (
        jax.ShapeDtypeStruct((_B, 12), jnp.float32),
        jax.ShapeDtypeStruct((_B, _NT), jnp.float32),
        jax.ShapeDtypeStruct((_B, 2), jnp.float32),
        jax.ShapeDtypeStruct((_B, 128), jnp.float32),
        jax.ShapeDtypeStruct((_B, 1), jnp.int32),
    )
    out_specs = (_rows(12), _rows(_NT), _rows(2), _rows(128), _rows(1))
    outs = pl.pallas_call(
        _tc_body,
        grid=(grid,),
        in_specs=in_specs,
        out_specs=out_specs,
        out_shape=out_shapes,
        compiler_params=pltpu.CompilerParams(
            dimension_semantics=("arbitrary",),
        ),
    )(stf, params, *args)
    return outs + (maskf,)


def kernel(params, surface_type, type_emb, W_pe, b_pe,
           enc_W1, enc_b1, enc_W2, enc_b2, enc_W3, enc_b3, enc_W4, enc_b4,
           fsq_Win, fsq_bin, fsq_Wout, fsq_bout,
           dec_W1, dec_b1, dec_W2, dec_b2, dec_W3, dec_b3,
           cls_W, cls_b, isc_W, isc_b, decraw_W, decraw_b):
    stf = surface_type.astype(jnp.int32).reshape(_B, 1)
    args = (
        W_pe.reshape(_NT * 32, 12).T,
        b_pe.reshape(1, _NT * 32),
        enc_W1[:, :32].T.astype(jnp.bfloat16), enc_W1[:, 32:].T.astype(jnp.bfloat16), enc_b1.reshape(1, -1),
        enc_W2.T.astype(jnp.bfloat16), enc_b2.reshape(1, -1),
        enc_W3.T.astype(jnp.bfloat16), enc_b3.reshape(1, -1),
        enc_W4.T.astype(jnp.bfloat16), enc_b4.reshape(1, -1),
        fsq_Win.T, fsq_bin.reshape(1, -1),
        fsq_Wout.T, fsq_bout.reshape(1, -1),
        cls_W.T, cls_b.reshape(1, -1),
        isc_W.T, isc_b.reshape(1, -1),
        dec_W1[:, :128].T.astype(jnp.bfloat16), dec_W1[:, 128:].T.astype(jnp.bfloat16), dec_b1.reshape(1, -1),
        dec_W2.T.astype(jnp.bfloat16), dec_b2.reshape(1, -1),
        dec_W3.T.astype(jnp.bfloat16), dec_b3.reshape(1, -1),
        decraw_W.reshape(_NT * 12, 32).T,
        decraw_b.reshape(1, _NT * 12),
        type_emb,
        jnp.asarray(_SHIFT).reshape(1, 4), jnp.asarray(_HALF_L).reshape(1, 4),
        jnp.asarray(_OFFSET).reshape(1, 4), jnp.asarray(_HALF_W).reshape(1, 4),
        jnp.asarray(_BASIS).reshape(1, 4),
    )
    recon, cls, isc, zq, idx, maskf = _run(stf, params, args)
    mask = maskf.T > 0.5
    return recon, mask, cls, isc, zq, idx.reshape(_B)


# speedup vs baseline: 1.0084x; 1.0084x over previous
import functools

import jax
import jax.numpy as jnp
import numpy as np
from jax import lax
from jax.experimental import pallas as pl
from jax.experimental.pallas import tpu as pltpu
from jax.experimental.pallas import tpu_sc as plsc

_LEVELS = np.array([8, 5, 5, 5])
_RAW_DIMS = np.array([7, 9, 10, 11, 12])
_B = 16384
_R = 1024
_NT = 5
_NC = 2
_NS = 16
_NW = _NC * _NS
_BPW = _B // _NW
_TD = 16

_EPS = 1e-3
_HALF_L = ((_LEVELS - 1.0) * (1.0 + _EPS) / 2.0).astype(np.float32)
_OFFSET = np.where(_LEVELS % 2 == 0, 0.5, 0.0).astype(np.float32)
_SHIFT = np.arctanh(_OFFSET / _HALF_L).astype(np.float32)
_HALF_W = (_LEVELS // 2).astype(np.float32)
_BASIS = np.concatenate([[1], np.cumprod(_LEVELS[:-1])]).astype(np.float32)
_MASK_TABLE = (np.arange(12)[None, :] < _RAW_DIMS[:, None]).astype(np.float32)


def _sc_body(st_hbm, out_hbm, st_v, cols_v):
    wid = lax.axis_index("s") * _NC + lax.axis_index("c")
    base = wid * _BPW
    pltpu.sync_copy(st_hbm.at[pl.ds(base, _BPW)], st_v)

    def body(g):
        st16 = st_v[pl.ds(g * 16, 16)]
        rd16 = jnp.full((16,), int(_RAW_DIMS[0]), jnp.int32)
        for t in range(1, _NT):
            d = st16 - t
            eq = 1 - jnp.minimum(d * d, 1)
            rd16 = rd16 + eq * int(_RAW_DIMS[t] - _RAW_DIMS[0])
        for c in range(12):
            col = jnp.minimum(jnp.maximum(rd16 - c, 0), 1)
            cols_v[c, pl.ds(g * 16, 16)] = col.astype(jnp.float32)

    for g in range(_BPW // 16):
        body(g)
    for c in range(12):
        pltpu.sync_copy(cols_v.at[c], out_hbm.at[c, pl.ds(base, _BPW)])


_sc_route_mask = functools.partial(
    pl.kernel,
    out_type=jax.ShapeDtypeStruct((12, _B), jnp.float32),
    mesh=plsc.VectorSubcoreMesh(core_axis_name="c", subcore_axis_name="s"),
    scratch_types=[
        pltpu.VMEM((_BPW,), jnp.int32),
        pltpu.VMEM((12, _BPW), jnp.float32),
    ],
)(_sc_body)


def _tc_body(stf_ref, params_ref,
             wpeT_ref, bpe_ref,
             w1aT_ref, w1bT_ref, b1_ref,
             w2T_ref, b2_ref, w3T_ref, b3_ref, w4T_ref, b4_ref,
             fwinT_ref, fbin_ref, fwoutT_ref, fbout_ref,
             clsT_ref, clsb_ref, iscT_ref, iscb_ref,
             d1aT_ref, d1bT_ref, db1_ref, d2T_ref, db2_ref, d3T_ref, db3_ref,
             wdrT_ref, bdr_ref, temb_ref,
             shift_ref, halfl_ref, offs_ref, halfw_ref, basis_ref,
             recon_ref, cls_ref, isc_ref, zq_ref, idx_ref):
    f32 = jnp.float32
    dot = functools.partial(jnp.dot, preferred_element_type=f32)
    sti = stf_ref[...]
    iota5 = jax.lax.broadcasted_iota(jnp.int32, (_R, _NT), 1)
    onehot = (iota5 == sti).astype(f32)
    emb = dot(onehot, temb_ref[...])

    p5 = dot(params_ref[...], wpeT_ref[...]) + bpe_ref[...]
    pe = onehot[:, 0:1] * p5[:, 0:32]
    for t in range(1, _NT):
        pe = pe + onehot[:, t:t + 1] * p5[:, 32 * t:32 * (t + 1)]

    h = jnp.maximum(dot(pe, w1aT_ref[...]) + dot(emb, w1bT_ref[...]) + b1_ref[...], 0.0)
    h = jnp.maximum(dot(h, w2T_ref[...]) + b2_ref[...], 0.0)
    h = jnp.maximum(dot(h, w3T_ref[...]) + b3_ref[...], 0.0)
    z = dot(h, w4T_ref[...]) + b4_ref[...]

    zp = dot(z, fwinT_ref[...]) + fbin_ref[...]
    bounded = jnp.tanh(zp + shift_ref[...]) * halfl_ref[...] - offs_ref[...]
    rounded = jnp.round(bounded)
    codes = rounded / halfw_ref[...]
    idx_f = jnp.sum((rounded + halfw_ref[...]) * basis_ref[...],
                    axis=1, keepdims=True)
    idx_ref[...] = idx_f.astype(jnp.int32)
    zq = dot(codes, fwoutT_ref[...]) + fbout_ref[...]
    zq_ref[...] = zq

    cls_ref[...] = dot(zq, clsT_ref[...]) + clsb_ref[...]
    isc_ref[...] = dot(zq, iscT_ref[...]) + iscb_ref[...]

    hd = jnp.maximum(dot(zq, d1aT_ref[...]) + dot(emb, d1bT_ref[...]) + db1_ref[...], 0.0)
    hd = jnp.maximum(dot(hd, d2T_ref[...]) + db2_ref[...], 0.0)
    pd = dot(hd, d3T_ref[...]) + db3_ref[...]

    d5 = dot(pd, wdrT_ref[...]) + bdr_ref[...]
    recon = onehot[:, 0:1] * d5[:, 0:12]
    for t in range(1, _NT):
        recon = recon + onehot[:, t:t + 1] * d5[:, 12 * t:12 * (t + 1)]
    recon_ref[...] = recon


def _full(shape):
    nd = len(shape)
    return pl.BlockSpec(shape, lambda i: (0,) * nd)


def _rows(width):
    return pl.BlockSpec((_R, width), lambda i: (i, 0))


@jax.jit
def _run(stf, params, args):
    maskf = _sc_route_mask(stf.reshape(_B))
    grid = _B // _R
    in_specs = [_rows(1), _rows(12)] + [_full(a.shape) for a in args]
    out_shapes = (
        jax.ShapeDtypeStruct((_B, 12), jnp.float32),
        jax.ShapeDtypeStruct((_B, _NT), jnp.float32),
        jax.ShapeDtypeStruct((_B, 2), jnp.float32),
        jax.ShapeDtypeStruct((_B, 128), jnp.float32),
        jax.ShapeDtypeStruct((_B, 1), jnp.int32),
    )
    out_specs = (_rows(12), _rows(_NT), _rows(2), _rows(128), _rows(1))
    outs = pl.pallas_call(
        _tc_body,
        grid=(grid,),
        in_specs=in_specs,
        out_specs=out_specs,
        out_shape=out_shapes,
        compiler_params=pltpu.CompilerParams(
            dimension_semantics=("parallel",),
        ),
    )(stf, params, *args)
    return outs + (maskf,)


def kernel(params, surface_type, type_emb, W_pe, b_pe,
           enc_W1, enc_b1, enc_W2, enc_b2, enc_W3, enc_b3, enc_W4, enc_b4,
           fsq_Win, fsq_bin, fsq_Wout, fsq_bout,
           dec_W1, dec_b1, dec_W2, dec_b2, dec_W3, dec_b3,
           cls_W, cls_b, isc_W, isc_b, decraw_W, decraw_b):
    stf = surface_type.astype(jnp.int32).reshape(_B, 1)
    args = (
        W_pe.reshape(_NT * 32, 12).T,
        b_pe.reshape(1, _NT * 32),
        enc_W1[:, :32].T, enc_W1[:, 32:].T, enc_b1.reshape(1, -1),
        enc_W2.T, enc_b2.reshape(1, -1),
        enc_W3.T, enc_b3.reshape(1, -1),
        enc_W4.T, enc_b4.reshape(1, -1),
        fsq_Win.T, fsq_bin.reshape(1, -1),
        fsq_Wout.T, fsq_bout.reshape(1, -1),
        cls_W.T, cls_b.reshape(1, -1),
        isc_W.T, isc_b.reshape(1, -1),
        dec_W1[:, :128].T, dec_W1[:, 128:].T, dec_b1.reshape(1, -1),
        dec_W2.T, dec_b2.reshape(1, -1),
        dec_W3.T, dec_b3.reshape(1, -1),
        decraw_W.reshape(_NT * 12, 32).T,
        decraw_b.reshape(1, _NT * 12),
        type_emb,
        jnp.asarray(_SHIFT).reshape(1, 4), jnp.asarray(_HALF_L).reshape(1, 4),
        jnp.asarray(_OFFSET).reshape(1, 4), jnp.asarray(_HALF_W).reshape(1, 4),
        jnp.asarray(_BASIS).reshape(1, 4),
    )
    recon, cls, isc, zq, idx, maskf = _run(stf, params, args)
    mask = maskf.T > 0.5
    return recon, mask, cls, isc, zq, idx.reshape(_B)


# speedup vs baseline: 1.0634x; 1.0545x over previous
import functools

import jax
import jax.numpy as jnp
import numpy as np
from jax import lax
from jax.experimental import pallas as pl
from jax.experimental.pallas import tpu as pltpu
from jax.experimental.pallas import tpu_sc as plsc

_LEVELS = np.array([8, 5, 5, 5])
_RAW_DIMS = np.array([7, 9, 10, 11, 12])
_B = 16384
_R = 2048
_NT = 5
_NC = 2
_NS = 16
_NW = _NC * _NS
_BPW = _B // _NW
_TD = 16

_EPS = 1e-3
_HALF_L = ((_LEVELS - 1.0) * (1.0 + _EPS) / 2.0).astype(np.float32)
_OFFSET = np.where(_LEVELS % 2 == 0, 0.5, 0.0).astype(np.float32)
_SHIFT = np.arctanh(_OFFSET / _HALF_L).astype(np.float32)
_HALF_W = (_LEVELS // 2).astype(np.float32)
_BASIS = np.concatenate([[1], np.cumprod(_LEVELS[:-1])]).astype(np.float32)
_MASK_TABLE = (np.arange(12)[None, :] < _RAW_DIMS[:, None]).astype(np.float32)


def _sc_body(st_hbm, out_hbm, st_v, cols_v):
    wid = lax.axis_index("s") * _NC + lax.axis_index("c")
    base = wid * _BPW
    pltpu.sync_copy(st_hbm.at[pl.ds(base, _BPW)], st_v)

    def body(g):
        st16 = st_v[pl.ds(g * 16, 16)]
        rd16 = jnp.full((16,), int(_RAW_DIMS[0]), jnp.int32)
        for t in range(1, _NT):
            d = st16 - t
            eq = 1 - jnp.minimum(d * d, 1)
            rd16 = rd16 + eq * int(_RAW_DIMS[t] - _RAW_DIMS[0])
        for c in range(12):
            col = jnp.minimum(jnp.maximum(rd16 - c, 0), 1)
            cols_v[c, pl.ds(g * 16, 16)] = col.astype(jnp.float32)

    for g in range(_BPW // 16):
        body(g)
    for c in range(12):
        pltpu.sync_copy(cols_v.at[c], out_hbm.at[c, pl.ds(base, _BPW)])


_sc_route_mask = functools.partial(
    pl.kernel,
    out_type=jax.ShapeDtypeStruct((12, _B), jnp.float32),
    mesh=plsc.VectorSubcoreMesh(core_axis_name="c", subcore_axis_name="s"),
    scratch_types=[
        pltpu.VMEM((_BPW,), jnp.int32),
        pltpu.VMEM((12, _BPW), jnp.float32),
    ],
)(_sc_body)


def _tc_body(stf_ref, params_ref,
             wpeT_ref, bpe_ref,
             w1aT_ref, w1bT_ref, b1_ref,
             w2T_ref, b2_ref, w3T_ref, b3_ref, w4T_ref, b4_ref,
             fwinT_ref, fbin_ref, fwoutT_ref, fbout_ref,
             clsT_ref, clsb_ref, iscT_ref, iscb_ref,
             d1aT_ref, d1bT_ref, db1_ref, d2T_ref, db2_ref, d3T_ref, db3_ref,
             wdrT_ref, bdr_ref, temb_ref,
             shift_ref, halfl_ref, offs_ref, halfw_ref, basis_ref,
             recon_ref, cls_ref, isc_ref, zq_ref, idx_ref):
    f32 = jnp.float32
    dot = functools.partial(jnp.dot, preferred_element_type=f32)
    sti = stf_ref[...]
    iota5 = jax.lax.broadcasted_iota(jnp.int32, (_R, _NT), 1)
    onehot = (iota5 == sti).astype(f32)
    emb = dot(onehot, temb_ref[...])

    p5 = dot(params_ref[...], wpeT_ref[...]) + bpe_ref[...]
    pe = onehot[:, 0:1] * p5[:, 0:32]
    for t in range(1, _NT):
        pe = pe + onehot[:, t:t + 1] * p5[:, 32 * t:32 * (t + 1)]

    h = jnp.maximum(dot(pe, w1aT_ref[...]) + dot(emb, w1bT_ref[...]) + b1_ref[...], 0.0)
    h = jnp.maximum(dot(h, w2T_ref[...]) + b2_ref[...], 0.0)
    h = jnp.maximum(dot(h, w3T_ref[...]) + b3_ref[...], 0.0)
    z = dot(h, w4T_ref[...]) + b4_ref[...]

    zp = dot(z, fwinT_ref[...]) + fbin_ref[...]
    bounded = jnp.tanh(zp + shift_ref[...]) * halfl_ref[...] - offs_ref[...]
    rounded = jnp.round(bounded)
    codes = rounded / halfw_ref[...]
    idx_f = jnp.sum((rounded + halfw_ref[...]) * basis_ref[...],
                    axis=1, keepdims=True)
    idx_ref[...] = idx_f.astype(jnp.int32)
    zq = dot(codes, fwoutT_ref[...]) + fbout_ref[...]
    zq_ref[...] = zq

    cls_ref[...] = dot(zq, clsT_ref[...]) + clsb_ref[...]
    isc_ref[...] = dot(zq, iscT_ref[...]) + iscb_ref[...]

    hd = jnp.maximum(dot(zq, d1aT_ref[...]) + dot(emb, d1bT_ref[...]) + db1_ref[...], 0.0)
    hd = jnp.maximum(dot(hd, d2T_ref[...]) + db2_ref[...], 0.0)
    pd = dot(hd, d3T_ref[...]) + db3_ref[...]

    d5 = dot(pd, wdrT_ref[...]) + bdr_ref[...]
    recon = onehot[:, 0:1] * d5[:, 0:12]
    for t in range(1, _NT):
        recon = recon + onehot[:, t:t + 1] * d5[:, 12 * t:12 * (t + 1)]
    recon_ref[...] = recon


def _full(shape):
    nd = len(shape)
    return pl.BlockSpec(shape, lambda i: (0,) * nd)


def _rows(width):
    return pl.BlockSpec((_R, width), lambda i: (i, 0))


@jax.jit
def _run(stf, params, args):
    maskf = _sc_route_mask(stf.reshape(_B))
    grid = _B // _R
    in_specs = [_rows(1), _rows(12)] + [_full(a.shape) for a in args]
    out_shapes = (
        jax.ShapeDtypeStruct((_B, 12), jnp.float32),
        jax.ShapeDtypeStruct((_B, _NT), jnp.float32),
        jax.ShapeDtypeStruct((_B, 2), jnp.float32),
        jax.ShapeDtypeStruct((_B, 128), jnp.float32),
        jax.ShapeDtypeStruct((_B, 1), jnp.int32),
    )
    out_specs = (_rows(12), _rows(_NT), _rows(2), _rows(128), _rows(1))
    outs = pl.pallas_call(
        _tc_body,
        grid=(grid,),
        in_specs=in_specs,
        out_specs=out_specs,
        out_shape=out_shapes,
        compiler_params=pltpu.CompilerParams(
            dimension_semantics=("parallel",),
        ),
    )(stf, params, *args)
    return outs + (maskf,)


def kernel(params, surface_type, type_emb, W_pe, b_pe,
           enc_W1, enc_b1, enc_W2, enc_b2, enc_W3, enc_b3, enc_W4, enc_b4,
           fsq_Win, fsq_bin, fsq_Wout, fsq_bout,
           dec_W1, dec_b1, dec_W2, dec_b2, dec_W3, dec_b3,
           cls_W, cls_b, isc_W, isc_b, decraw_W, decraw_b):
    stf = surface_type.astype(jnp.int32).reshape(_B, 1)
    args = (
        W_pe.reshape(_NT * 32, 12).T,
        b_pe.reshape(1, _NT * 32),
        enc_W1[:, :32].T, enc_W1[:, 32:].T, enc_b1.reshape(1, -1),
        enc_W2.T, enc_b2.reshape(1, -1),
        enc_W3.T, enc_b3.reshape(1, -1),
        enc_W4.T, enc_b4.reshape(1, -1),
        fsq_Win.T, fsq_bin.reshape(1, -1),
        fsq_Wout.T, fsq_bout.reshape(1, -1),
        cls_W.T, cls_b.reshape(1, -1),
        isc_W.T, isc_b.reshape(1, -1),
        dec_W1[:, :128].T, dec_W1[:, 128:].T, dec_b1.reshape(1, -1),
        dec_W2.T, dec_b2.reshape(1, -1),
        dec_W3.T, dec_b3.reshape(1, -1),
        decraw_W.reshape(_NT * 12, 32).T,
        decraw_b.reshape(1, _NT * 12),
        type_emb,
        jnp.asarray(_SHIFT).reshape(1, 4), jnp.asarray(_HALF_L).reshape(1, 4),
        jnp.asarray(_OFFSET).reshape(1, 4), jnp.asarray(_HALF_W).reshape(1, 4),
        jnp.asarray(_BASIS).reshape(1, 4),
    )
    recon, cls, isc, zq, idx, maskf = _run(stf, params, args)
    mask = maskf.T > 0.5
    return recon, mask, cls, isc, zq, idx.reshape(_B)


# speedup vs baseline: 1.0669x; 1.0034x over previous
import functools

import jax
import jax.numpy as jnp
import numpy as np
from jax import lax
from jax.experimental import pallas as pl
from jax.experimental.pallas import tpu as pltpu
from jax.experimental.pallas import tpu_sc as plsc

_LEVELS = np.array([8, 5, 5, 5])
_RAW_DIMS = np.array([7, 9, 10, 11, 12])
_B = 16384
_R = 4096
_NT = 5
_NC = 2
_NS = 16
_NW = _NC * _NS
_BPW = _B // _NW
_TD = 16

_EPS = 1e-3
_HALF_L = ((_LEVELS - 1.0) * (1.0 + _EPS) / 2.0).astype(np.float32)
_OFFSET = np.where(_LEVELS % 2 == 0, 0.5, 0.0).astype(np.float32)
_SHIFT = np.arctanh(_OFFSET / _HALF_L).astype(np.float32)
_HALF_W = (_LEVELS // 2).astype(np.float32)
_BASIS = np.concatenate([[1], np.cumprod(_LEVELS[:-1])]).astype(np.float32)
_MASK_TABLE = (np.arange(12)[None, :] < _RAW_DIMS[:, None]).astype(np.float32)


def _sc_body(st_hbm, out_hbm, st_v, cols_v):
    wid = lax.axis_index("s") * _NC + lax.axis_index("c")
    base = wid * _BPW
    pltpu.sync_copy(st_hbm.at[pl.ds(base, _BPW)], st_v)

    def body(g):
        st16 = st_v[pl.ds(g * 16, 16)]
        rd16 = jnp.full((16,), int(_RAW_DIMS[0]), jnp.int32)
        for t in range(1, _NT):
            d = st16 - t
            eq = 1 - jnp.minimum(d * d, 1)
            rd16 = rd16 + eq * int(_RAW_DIMS[t] - _RAW_DIMS[0])
        for c in range(12):
            col = jnp.minimum(jnp.maximum(rd16 - c, 0), 1)
            cols_v[c, pl.ds(g * 16, 16)] = col.astype(jnp.float32)

    for g in range(_BPW // 16):
        body(g)
    for c in range(12):
        pltpu.sync_copy(cols_v.at[c], out_hbm.at[c, pl.ds(base, _BPW)])


_sc_route_mask = functools.partial(
    pl.kernel,
    out_type=jax.ShapeDtypeStruct((12, _B), jnp.float32),
    mesh=plsc.VectorSubcoreMesh(core_axis_name="c", subcore_axis_name="s"),
    scratch_types=[
        pltpu.VMEM((_BPW,), jnp.int32),
        pltpu.VMEM((12, _BPW), jnp.float32),
    ],
)(_sc_body)


def _tc_body(stf_ref, params_ref,
             wpeT_ref, bpe_ref,
             w1aT_ref, w1bT_ref, b1_ref,
             w2T_ref, b2_ref, w3T_ref, b3_ref, w4T_ref, b4_ref,
             fwinT_ref, fbin_ref, fwoutT_ref, fbout_ref,
             clsT_ref, clsb_ref, iscT_ref, iscb_ref,
             d1aT_ref, d1bT_ref, db1_ref, d2T_ref, db2_ref, d3T_ref, db3_ref,
             wdrT_ref, bdr_ref, temb_ref,
             shift_ref, halfl_ref, offs_ref, halfw_ref, basis_ref,
             recon_ref, cls_ref, isc_ref, zq_ref, idx_ref):
    f32 = jnp.float32
    dot = functools.partial(jnp.dot, preferred_element_type=f32)
    sti = stf_ref[...]
    iota5 = jax.lax.broadcasted_iota(jnp.int32, (_R, _NT), 1)
    onehot = (iota5 == sti).astype(f32)
    emb = dot(onehot, temb_ref[...])

    p5 = dot(params_ref[...], wpeT_ref[...]) + bpe_ref[...]
    pe = onehot[:, 0:1] * p5[:, 0:32]
    for t in range(1, _NT):
        pe = pe + onehot[:, t:t + 1] * p5[:, 32 * t:32 * (t + 1)]

    h = jnp.maximum(dot(pe, w1aT_ref[...]) + dot(emb, w1bT_ref[...]) + b1_ref[...], 0.0)
    h = jnp.maximum(dot(h, w2T_ref[...]) + b2_ref[...], 0.0)
    h = jnp.maximum(dot(h, w3T_ref[...]) + b3_ref[...], 0.0)
    z = dot(h, w4T_ref[...]) + b4_ref[...]

    zp = dot(z, fwinT_ref[...]) + fbin_ref[...]
    bounded = jnp.tanh(zp + shift_ref[...]) * halfl_ref[...] - offs_ref[...]
    rounded = jnp.round(bounded)
    codes = rounded / halfw_ref[...]
    idx_f = jnp.sum((rounded + halfw_ref[...]) * basis_ref[...],
                    axis=1, keepdims=True)
    idx_ref[...] = idx_f.astype(jnp.int32)
    zq = dot(codes, fwoutT_ref[...]) + fbout_ref[...]
    zq_ref[...] = zq

    cls_ref[...] = dot(zq, clsT_ref[...]) + clsb_ref[...]
    isc_ref[...] = dot(zq, iscT_ref[...]) + iscb_ref[...]

    hd = jnp.maximum(dot(zq, d1aT_ref[...]) + dot(emb, d1bT_ref[...]) + db1_ref[...], 0.0)
    hd = jnp.maximum(dot(hd, d2T_ref[...]) + db2_ref[...], 0.0)
    pd = dot(hd, d3T_ref[...]) + db3_ref[...]

    d5 = dot(pd, wdrT_ref[...]) + bdr_ref[...]
    recon = onehot[:, 0:1] * d5[:, 0:12]
    for t in range(1, _NT):
        recon = recon + onehot[:, t:t + 1] * d5[:, 12 * t:12 * (t + 1)]
    recon_ref[...] = recon


def _full(shape):
    nd = len(shape)
    return pl.BlockSpec(shape, lambda i: (0,) * nd)


def _rows(width):
    return pl.BlockSpec((_R, width), lambda i: (i, 0))


@jax.jit
def _run(stf, params, args):
    maskf = _sc_route_mask(stf.reshape(_B))
    grid = _B // _R
    in_specs = [_rows(1), _rows(12)] + [_full(a.shape) for a in args]
    out_shapes = (
        jax.ShapeDtypeStruct((_B, 12), jnp.float32),
        jax.ShapeDtypeStruct((_B, _NT), jnp.float32),
        jax.ShapeDtypeStruct((_B, 2), jnp.float32),
        jax.ShapeDtypeStruct((_B, 128), jnp.float32),
        jax.ShapeDtypeStruct((_B, 1), jnp.int32),
    )
    out_specs = (_rows(12), _rows(_NT), _rows(2), _rows(128), _rows(1))
    outs = pl.pallas_call(
        _tc_body,
        grid=(grid,),
        in_specs=in_specs,
        out_specs=out_specs,
        out_shape=out_shapes,
        compiler_params=pltpu.CompilerParams(
            dimension_semantics=("parallel",),
        ),
    )(stf, params, *args)
    return outs + (maskf,)


def kernel(params, surface_type, type_emb, W_pe, b_pe,
           enc_W1, enc_b1, enc_W2, enc_b2, enc_W3, enc_b3, enc_W4, enc_b4,
           fsq_Win, fsq_bin, fsq_Wout, fsq_bout,
           dec_W1, dec_b1, dec_W2, dec_b2, dec_W3, dec_b3,
           cls_W, cls_b, isc_W, isc_b, decraw_W, decraw_b):
    stf = surface_type.astype(jnp.int32).reshape(_B, 1)
    args = (
        W_pe.reshape(_NT * 32, 12).T,
        b_pe.reshape(1, _NT * 32),
        enc_W1[:, :32].T, enc_W1[:, 32:].T, enc_b1.reshape(1, -1),
        enc_W2.T, enc_b2.reshape(1, -1),
        enc_W3.T, enc_b3.reshape(1, -1),
        enc_W4.T, enc_b4.reshape(1, -1),
        fsq_Win.T, fsq_bin.reshape(1, -1),
        fsq_Wout.T, fsq_bout.reshape(1, -1),
        cls_W.T, cls_b.reshape(1, -1),
        isc_W.T, isc_b.reshape(1, -1),
        dec_W1[:, :128].T, dec_W1[:, 128:].T, dec_b1.reshape(1, -1),
        dec_W2.T, dec_b2.reshape(1, -1),
        dec_W3.T, dec_b3.reshape(1, -1),
        decraw_W.reshape(_NT * 12, 32).T,
        decraw_b.reshape(1, _NT * 12),
        type_emb,
        jnp.asarray(_SHIFT).reshape(1, 4), jnp.asarray(_HALF_L).reshape(1, 4),
        jnp.asarray(_OFFSET).reshape(1, 4), jnp.asarray(_HALF_W).reshape(1, 4),
        jnp.asarray(_BASIS).reshape(1, 4),
    )
    recon, cls, isc, zq, idx, maskf = _run(stf, params, args)
    mask = maskf.T > 0.5
    return recon, mask, cls, isc, zq, idx.reshape(_B)


# speedup vs baseline: 1.1382x; 1.0668x over previous
import functools

import jax
import jax.numpy as jnp
import numpy as np
from jax import lax
from jax.experimental import pallas as pl
from jax.experimental.pallas import tpu as pltpu
from jax.experimental.pallas import tpu_sc as plsc

_LEVELS = np.array([8, 5, 5, 5])
_RAW_DIMS = np.array([7, 9, 10, 11, 12])
_B = 16384
_R = 4096
_NT = 5
_NC = 2
_NS = 16
_NW = _NC * _NS
_BPW = _B // _NW
_TD = 16

_EPS = 1e-3
_HALF_L = ((_LEVELS - 1.0) * (1.0 + _EPS) / 2.0).astype(np.float32)
_OFFSET = np.where(_LEVELS % 2 == 0, 0.5, 0.0).astype(np.float32)
_SHIFT = np.arctanh(_OFFSET / _HALF_L).astype(np.float32)
_HALF_W = (_LEVELS // 2).astype(np.float32)
_BASIS = np.concatenate([[1], np.cumprod(_LEVELS[:-1])]).astype(np.float32)
_MASK_TABLE = (np.arange(12)[None, :] < _RAW_DIMS[:, None]).astype(np.float32)


def _sc_body(st_hbm, out_hbm, st_v, cols_v):
    wid = lax.axis_index("s") * _NC + lax.axis_index("c")
    base = wid * _BPW
    pltpu.sync_copy(st_hbm.at[pl.ds(base, _BPW)], st_v)

    def body(g):
        st16 = st_v[pl.ds(g * 16, 16)]
        rd16 = jnp.full((16,), int(_RAW_DIMS[0]), jnp.int32)
        for t in range(1, _NT):
            d = st16 - t
            eq = 1 - jnp.minimum(d * d, 1)
            rd16 = rd16 + eq * int(_RAW_DIMS[t] - _RAW_DIMS[0])
        for c in range(12):
            col = jnp.minimum(jnp.maximum(rd16 - c, 0), 1)
            cols_v[c, pl.ds(g * 16, 16)] = col.astype(jnp.float32)

    for g in range(_BPW // 16):
        body(g)
    for c in range(12):
        pltpu.sync_copy(cols_v.at[c], out_hbm.at[c, pl.ds(base, _BPW)])


_sc_route_mask = functools.partial(
    pl.kernel,
    out_type=jax.ShapeDtypeStruct((12, _B), jnp.float32),
    mesh=plsc.VectorSubcoreMesh(core_axis_name="c", subcore_axis_name="s"),
    scratch_types=[
        pltpu.VMEM((_BPW,), jnp.int32),
        pltpu.VMEM((12, _BPW), jnp.float32),
    ],
)(_sc_body)


def _tc_body(stf_ref, params_ref,
             wpe_ref, bpe_ref,
             w1_ref, b1_ref, w2_ref, b2_ref, w3_ref, b3_ref, w4_ref, b4_ref,
             fwin_ref, fbin_ref, fwout_ref, fbout_ref,
             cls_ref_w, clsb_ref, isc_ref_w, iscb_ref,
             d1_ref, db1_ref, d2_ref, db2_ref, d3_ref, db3_ref,
             wdr_ref, bdr_ref, temb_ref,
             shift_ref, halfl_ref, offs_ref, halfw_ref, basis_ref,
             recon_ref, cls_ref, isc_ref, zq_ref, idx_ref):
    f32 = jnp.float32

    def dg(x, w):
        return lax.dot_general(x, w, (((1,), (1,)), ((), ())),
                               preferred_element_type=f32)

    sti = stf_ref[...]
    iota5 = jax.lax.broadcasted_iota(jnp.int32, (_R, _NT), 1)
    onehot = (iota5 == sti).astype(f32)
    emb = jnp.dot(onehot, temb_ref[...], preferred_element_type=f32)

    p5 = dg(params_ref[...], wpe_ref[...].reshape(_NT * 32, 12)) + bpe_ref[...][None, :]
    pe = onehot[:, 0:1] * p5[:, 0:32]
    for t in range(1, _NT):
        pe = pe + onehot[:, t:t + 1] * p5[:, 32 * t:32 * (t + 1)]

    w1 = w1_ref[...]
    h = jnp.maximum(dg(pe, w1[:, :32]) + dg(emb, w1[:, 32:]) + b1_ref[...][None, :], 0.0)
    h = jnp.maximum(dg(h, w2_ref[...]) + b2_ref[...][None, :], 0.0)
    h = jnp.maximum(dg(h, w3_ref[...]) + b3_ref[...][None, :], 0.0)
    z = dg(h, w4_ref[...]) + b4_ref[...][None, :]

    zp = dg(z, fwin_ref[...]) + fbin_ref[...][None, :]
    bounded = jnp.tanh(zp + shift_ref[...]) * halfl_ref[...] - offs_ref[...]
    rounded = jnp.round(bounded)
    codes = rounded / halfw_ref[...]
    idx_f = jnp.sum((rounded + halfw_ref[...]) * basis_ref[...],
                    axis=1, keepdims=True)
    idx_ref[...] = idx_f.astype(jnp.int32)
    zq = dg(codes, fwout_ref[...]) + fbout_ref[...][None, :]
    zq_ref[...] = zq

    cls_ref[...] = dg(zq, cls_ref_w[...]) + clsb_ref[...][None, :]
    isc_ref[...] = dg(zq, isc_ref_w[...]) + iscb_ref[...][None, :]

    d1 = d1_ref[...]
    hd = jnp.maximum(dg(zq, d1[:, :128]) + dg(emb, d1[:, 128:]) + db1_ref[...][None, :], 0.0)
    hd = jnp.maximum(dg(hd, d2_ref[...]) + db2_ref[...][None, :], 0.0)
    pd = dg(hd, d3_ref[...]) + db3_ref[...][None, :]

    d5 = dg(pd, wdr_ref[...].reshape(_NT * 12, 32)) + bdr_ref[...][None, :]
    recon = onehot[:, 0:1] * d5[:, 0:12]
    for t in range(1, _NT):
        recon = recon + onehot[:, t:t + 1] * d5[:, 12 * t:12 * (t + 1)]
    recon_ref[...] = recon


def _full(shape):
    nd = len(shape)
    return pl.BlockSpec(shape, lambda i: (0,) * nd)


def _rows(width):
    return pl.BlockSpec((_R, width), lambda i: (i, 0))


@jax.jit
def _run(stf, params, args):
    maskf = _sc_route_mask(stf.reshape(_B))
    grid = _B // _R
    in_specs = [_rows(1), _rows(12)] + [_full(a.shape) for a in args]
    out_shapes = (
        jax.ShapeDtypeStruct((_B, 12), jnp.float32),
        jax.ShapeDtypeStruct((_B, _NT), jnp.float32),
        jax.ShapeDtypeStruct((_B, 2), jnp.float32),
        jax.ShapeDtypeStruct((_B, 128), jnp.float32),
        jax.ShapeDtypeStruct((_B, 1), jnp.int32),
    )
    out_specs = (_rows(12), _rows(_NT), _rows(2), _rows(128), _rows(1))
    outs = pl.pallas_call(
        _tc_body,
        grid=(grid,),
        in_specs=in_specs,
        out_specs=out_specs,
        out_shape=out_shapes,
        compiler_params=pltpu.CompilerParams(
            dimension_semantics=("parallel",),
        ),
    )(stf, params, *args)
    return outs + (maskf,)


def kernel(params, surface_type, type_emb, W_pe, b_pe,
           enc_W1, enc_b1, enc_W2, enc_b2, enc_W3, enc_b3, enc_W4, enc_b4,
           fsq_Win, fsq_bin, fsq_Wout, fsq_bout,
           dec_W1, dec_b1, dec_W2, dec_b2, dec_W3, dec_b3,
           cls_W, cls_b, isc_W, isc_b, decraw_W, decraw_b):
    stf = surface_type.astype(jnp.int32).reshape(_B, 1)
    args = (
        W_pe,
        b_pe.reshape(-1),
        enc_W1, enc_b1, enc_W2, enc_b2, enc_W3, enc_b3, enc_W4, enc_b4,
        fsq_Win, fsq_bin, fsq_Wout, fsq_bout,
        cls_W, cls_b, isc_W, isc_b,
        dec_W1, dec_b1, dec_W2, dec_b2, dec_W3, dec_b3,
        decraw_W,
        decraw_b.reshape(-1),
        type_emb,
        jnp.asarray(_SHIFT).reshape(1, 4), jnp.asarray(_HALF_L).reshape(1, 4),
        jnp.asarray(_OFFSET).reshape(1, 4), jnp.asarray(_HALF_W).reshape(1, 4),
        jnp.asarray(_BASIS).reshape(1, 4),
    )
    recon, cls, isc, zq, idx, maskf = _run(stf, params, args)
    mask = maskf.T > 0.5
    return recon, mask, cls, isc, zq, idx.reshape(_B)


# speedup vs baseline: 1.9021x; 1.6712x over previous
import functools

import jax
import jax.numpy as jnp
import numpy as np
from jax import lax
from jax.experimental import pallas as pl
from jax.experimental.pallas import tpu as pltpu
from jax.experimental.pallas import tpu_sc as plsc

_LEVELS = np.array([8, 5, 5, 5])
_RAW_DIMS = np.array([7, 9, 10, 11, 12])
_B = 16384
_R = 4096
_NT = 5
_NC = 2
_NS = 16
_NW = _NC * _NS
_BPW = _B // _NW
_TD = 16

_EPS = 1e-3
_HALF_L = ((_LEVELS - 1.0) * (1.0 + _EPS) / 2.0).astype(np.float32)
_OFFSET = np.where(_LEVELS % 2 == 0, 0.5, 0.0).astype(np.float32)
_SHIFT = np.arctanh(_OFFSET / _HALF_L).astype(np.float32)
_HALF_W = (_LEVELS // 2).astype(np.float32)
_BASIS = np.concatenate([[1], np.cumprod(_LEVELS[:-1])]).astype(np.float32)
_MASK_TABLE = (np.arange(12)[None, :] < _RAW_DIMS[:, None]).astype(np.float32)


def _sc_body(st_hbm, out_hbm, st_v, cols_v):
    wid = lax.axis_index("s") * _NC + lax.axis_index("c")
    base = wid * _BPW
    pltpu.sync_copy(st_hbm.at[pl.ds(base, _BPW)], st_v)

    def body(g):
        st16 = st_v[pl.ds(g * 16, 16)]
        rd16 = jnp.full((16,), int(_RAW_DIMS[0]), jnp.int32)
        for t in range(1, _NT):
            d = st16 - t
            eq = 1 - jnp.minimum(d * d, 1)
            rd16 = rd16 + eq * int(_RAW_DIMS[t] - _RAW_DIMS[0])
        for c in range(12):
            col = jnp.minimum(jnp.maximum(rd16 - c, 0), 1)
            cols_v[c, pl.ds(g * 16, 16)] = col.astype(jnp.float32)

    for g in range(_BPW // 16):
        body(g)
    for c in range(12):
        pltpu.sync_copy(cols_v.at[c], out_hbm.at[c, pl.ds(base, _BPW)])


_sc_route_mask = functools.partial(
    pl.kernel,
    out_type=jax.ShapeDtypeStruct((12, _B), jnp.float32),
    mesh=plsc.VectorSubcoreMesh(core_axis_name="c", subcore_axis_name="s"),
    scratch_types=[
        pltpu.VMEM((_BPW,), jnp.int32),
        pltpu.VMEM((12, _BPW), jnp.float32),
    ],
)(_sc_body)


def _tc_body(strow_ref, paramsT_ref,
             wpe_ref, bpe_ref,
             w1_ref, b1_ref, w2_ref, b2_ref, w3_ref, b3_ref, w4_ref, b4_ref,
             fwin_ref, fbin_ref, fwout_ref, fbout_ref,
             clsw_ref, clsb_ref, iscw_ref, iscb_ref,
             d1_ref, db1_ref, d2_ref, db2_ref, d3_ref, db3_ref,
             wdr_ref, bdr_ref, temb_ref,
             shift_ref, halfl_ref, offs_ref, halfw_ref, basis_ref,
             reconT_ref, clsT_ref, iscT_ref, zq_ref, idxT_ref):
    f32 = jnp.float32

    def dg(x, w):
        return lax.dot_general(x, w, (((1,), (1,)), ((), ())),
                               preferred_element_type=f32)

    sti = strow_ref[...]
    iotaT = jax.lax.broadcasted_iota(jnp.int32, (_NT, _R), 0)
    onehotT = (iotaT == sti).astype(f32)
    embT = lax.dot_general(temb_ref[...], onehotT,
                           (((0,), (0,)), ((), ())),
                           preferred_element_type=f32)

    p5T = lax.dot_general(wpe_ref[...].reshape(_NT * 32, 12), paramsT_ref[...],
                          (((1,), (0,)), ((), ())),
                          preferred_element_type=f32) + bpe_ref[...][:, None]
    peT = onehotT[0:1, :] * p5T[0:32, :]
    for t in range(1, _NT):
        peT = peT + onehotT[t:t + 1, :] * p5T[32 * t:32 * (t + 1), :]

    def dgT(xT, w):
        return lax.dot_general(xT, w, (((0,), (1,)), ((), ())),
                               preferred_element_type=f32)

    w1 = w1_ref[...]
    h = jnp.maximum(dgT(peT, w1[:, :32]) + dgT(embT, w1[:, 32:]) + b1_ref[...][None, :], 0.0)
    h = jnp.maximum(dg(h, w2_ref[...]) + b2_ref[...][None, :], 0.0)
    h = jnp.maximum(dg(h, w3_ref[...]) + b3_ref[...][None, :], 0.0)
    z = dg(h, w4_ref[...]) + b4_ref[...][None, :]

    zpT = lax.dot_general(fwin_ref[...], z, (((1,), (1,)), ((), ())),
                          preferred_element_type=f32) + fbin_ref[...][:, None]
    boundedT = jnp.tanh(zpT + shift_ref[...]) * halfl_ref[...] - offs_ref[...]
    roundedT = jnp.round(boundedT)
    codesT = roundedT / halfw_ref[...]
    idxT_f = jnp.sum((roundedT + halfw_ref[...]) * basis_ref[...],
                     axis=0, keepdims=True)
    idxT_ref[...] = idxT_f.astype(jnp.int32)
    zq = dgT(codesT, fwout_ref[...]) + fbout_ref[...][None, :]
    zq_ref[...] = zq

    clsT_ref[...] = lax.dot_general(clsw_ref[...], zq, (((1,), (1,)), ((), ())),
                                    preferred_element_type=f32) + clsb_ref[...][:, None]
    iscT_ref[...] = lax.dot_general(iscw_ref[...], zq, (((1,), (1,)), ((), ())),
                                    preferred_element_type=f32) + iscb_ref[...][:, None]

    d1 = d1_ref[...]
    hd = jnp.maximum(dg(zq, d1[:, :128]) + dgT(embT, d1[:, 128:]) + db1_ref[...][None, :], 0.0)
    hd = jnp.maximum(dg(hd, d2_ref[...]) + db2_ref[...][None, :], 0.0)
    pd = dg(hd, d3_ref[...]) + db3_ref[...][None, :]

    d5T = lax.dot_general(wdr_ref[...].reshape(_NT * 12, 32), pd,
                          (((1,), (1,)), ((), ())),
                          preferred_element_type=f32) + bdr_ref[...][:, None]
    reconT = onehotT[0:1, :] * d5T[0:12, :]
    for t in range(1, _NT):
        reconT = reconT + onehotT[t:t + 1, :] * d5T[12 * t:12 * (t + 1), :]
    reconT_ref[...] = reconT


def _full(shape):
    nd = len(shape)
    return pl.BlockSpec(shape, lambda i: (0,) * nd)


def _rows(width):
    return pl.BlockSpec((_R, width), lambda i: (i, 0))


@jax.jit
def _run(strow, paramsT, args):
    maskf = _sc_route_mask(strow.reshape(_B))
    grid = _B // _R
    in_specs = ([pl.BlockSpec((1, _R), lambda i: (0, i)),
                 pl.BlockSpec((12, _R), lambda i: (0, i))]
                + [_full(a.shape) for a in args])
    out_shapes = (
        jax.ShapeDtypeStruct((12, _B), jnp.float32),
        jax.ShapeDtypeStruct((_NT, _B), jnp.float32),
        jax.ShapeDtypeStruct((2, _B), jnp.float32),
        jax.ShapeDtypeStruct((_B, 128), jnp.float32),
        jax.ShapeDtypeStruct((1, _B), jnp.int32),
    )
    out_specs = (pl.BlockSpec((12, _R), lambda i: (0, i)),
                 pl.BlockSpec((_NT, _R), lambda i: (0, i)),
                 pl.BlockSpec((2, _R), lambda i: (0, i)),
                 pl.BlockSpec((_R, 128), lambda i: (i, 0)),
                 pl.BlockSpec((1, _R), lambda i: (0, i)))
    outs = pl.pallas_call(
        _tc_body,
        grid=(grid,),
        in_specs=in_specs,
        out_specs=out_specs,
        out_shape=out_shapes,
        compiler_params=pltpu.CompilerParams(
            dimension_semantics=("arbitrary",),
        ),
    )(strow, paramsT, *args)
    return outs + (maskf,)


def kernel(params, surface_type, type_emb, W_pe, b_pe,
           enc_W1, enc_b1, enc_W2, enc_b2, enc_W3, enc_b3, enc_W4, enc_b4,
           fsq_Win, fsq_bin, fsq_Wout, fsq_bout,
           dec_W1, dec_b1, dec_W2, dec_b2, dec_W3, dec_b3,
           cls_W, cls_b, isc_W, isc_b, decraw_W, decraw_b):
    strow = surface_type.astype(jnp.int32).reshape(1, _B)
    args = (
        W_pe,
        b_pe.reshape(-1),
        enc_W1, enc_b1, enc_W2, enc_b2, enc_W3, enc_b3, enc_W4, enc_b4,
        fsq_Win, fsq_bin, fsq_Wout, fsq_bout,
        cls_W, cls_b, isc_W, isc_b,
        dec_W1, dec_b1, dec_W2, dec_b2, dec_W3, dec_b3,
        decraw_W,
        decraw_b.reshape(-1),
        type_emb,
        jnp.asarray(_SHIFT).reshape(4, 1), jnp.asarray(_HALF_L).reshape(4, 1),
        jnp.asarray(_OFFSET).reshape(4, 1), jnp.asarray(_HALF_W).reshape(4, 1),
        jnp.asarray(_BASIS).reshape(4, 1),
    )
    reconT, clsT, iscT, zq, idxT, maskf = _run(strow, params.T, args)
    mask = maskf.T > 0.5
    return reconT.T, mask, clsT.T, iscT.T, zq, idxT.reshape(_B)


# speedup vs baseline: 2.1507x; 1.1307x over previous
import functools

import jax
import jax.numpy as jnp
import numpy as np
from jax import lax
from jax.experimental import pallas as pl
from jax.experimental.pallas import tpu as pltpu
from jax.experimental.pallas import tpu_sc as plsc

_LEVELS = np.array([8, 5, 5, 5])
_RAW_DIMS = np.array([7, 9, 10, 11, 12])
_B = 16384
_R = 4096
_NT = 5
_NC = 2
_NS = 16
_NW = _NC * _NS
_BPW = _B // _NW
_TD = 16

_EPS = 1e-3
_HALF_L = ((_LEVELS - 1.0) * (1.0 + _EPS) / 2.0).astype(np.float32)
_OFFSET = np.where(_LEVELS % 2 == 0, 0.5, 0.0).astype(np.float32)
_SHIFT = np.arctanh(_OFFSET / _HALF_L).astype(np.float32)
_HALF_W = (_LEVELS // 2).astype(np.float32)
_BASIS = np.concatenate([[1], np.cumprod(_LEVELS[:-1])]).astype(np.float32)
_MASK_TABLE = (np.arange(12)[None, :] < _RAW_DIMS[:, None]).astype(np.float32)


def _sc_body(st_hbm, out_hbm, st_v, cols_v):
    wid = lax.axis_index("s") * _NC + lax.axis_index("c")
    base = wid * _BPW
    pltpu.sync_copy(st_hbm.at[pl.ds(base, _BPW)], st_v)

    def body(g):
        st16 = st_v[pl.ds(g * 16, 16)]
        rd16 = jnp.full((16,), int(_RAW_DIMS[0]), jnp.int32)
        for t in range(1, _NT):
            d = st16 - t
            eq = 1 - jnp.minimum(d * d, 1)
            rd16 = rd16 + eq * int(_RAW_DIMS[t] - _RAW_DIMS[0])
        for c in range(12):
            col = jnp.minimum(jnp.maximum(rd16 - c, 0), 1)
            cols_v[c, pl.ds(g * 16, 16)] = col.astype(jnp.float32)

    for g in range(_BPW // 16):
        body(g)
    for c in range(12):
        pltpu.sync_copy(cols_v.at[c], out_hbm.at[c, pl.ds(base, _BPW)])


_sc_route_mask = functools.partial(
    pl.kernel,
    out_type=jax.ShapeDtypeStruct((12, _B), jnp.float32),
    mesh=plsc.VectorSubcoreMesh(core_axis_name="c", subcore_axis_name="s"),
    scratch_types=[
        pltpu.VMEM((_BPW,), jnp.int32),
        pltpu.VMEM((12, _BPW), jnp.float32),
    ],
)(_sc_body)


def _tc_body(strow_ref, paramsT_ref,
             wpe_ref, bpe_ref,
             w1_ref, b1_ref, w2_ref, b2_ref, w3_ref, b3_ref, w4_ref, b4_ref,
             fwin_ref, fbin_ref, fwout_ref, fbout_ref,
             clsw_ref, clsb_ref, iscw_ref, iscb_ref,
             d1_ref, db1_ref, d2_ref, db2_ref, d3_ref, db3_ref,
             wdr_ref, bdr_ref, temb_ref,
             shift_ref, halfl_ref, offs_ref, halfw_ref, basis_ref,
             reconT_ref, clsT_ref, iscT_ref, zq_ref, idxT_ref):
    f32 = jnp.float32

    def dg(x, w):
        return lax.dot_general(x, w, (((1,), (1,)), ((), ())),
                               preferred_element_type=f32)

    sti = strow_ref[...]
    iotaT = jax.lax.broadcasted_iota(jnp.int32, (_NT, _R), 0)
    onehotT = (iotaT == sti).astype(f32)
    embT = lax.dot_general(temb_ref[...], onehotT,
                           (((0,), (0,)), ((), ())),
                           preferred_element_type=f32)

    wpe = wpe_ref[...]
    pT = paramsT_ref[...]
    peT = onehotT[0:1, :] * lax.dot_general(
        wpe[0], pT, (((0,), (0,)), ((), ())), preferred_element_type=f32)
    for t in range(1, _NT):
        peT = peT + onehotT[t:t + 1, :] * lax.dot_general(
            wpe[t], pT, (((0,), (0,)), ((), ())), preferred_element_type=f32)
    peT = peT + lax.dot_general(bpe_ref[...], onehotT, (((0,), (0,)), ((), ())),
                                preferred_element_type=f32)

    def dgT(xT, w):
        return lax.dot_general(xT, w, (((0,), (1,)), ((), ())),
                               preferred_element_type=f32)

    w1T = w1_ref[...]
    h = jnp.maximum(
        lax.dot_general(peT, w1T[:32, :], (((0,), (0,)), ((), ())),
                        preferred_element_type=f32)
        + lax.dot_general(embT, w1T[32:, :], (((0,), (0,)), ((), ())),
                          preferred_element_type=f32)
        + b1_ref[...][None, :], 0.0)
    h = jnp.maximum(dg(h, w2_ref[...]) + b2_ref[...][None, :], 0.0)
    h = jnp.maximum(dg(h, w3_ref[...]) + b3_ref[...][None, :], 0.0)
    z = dg(h, w4_ref[...]) + b4_ref[...][None, :]

    zpT = lax.dot_general(fwin_ref[...], z, (((1,), (1,)), ((), ())),
                          preferred_element_type=f32) + fbin_ref[...][:, None]
    boundedT = jnp.tanh(zpT + shift_ref[...]) * halfl_ref[...] - offs_ref[...]
    roundedT = jnp.round(boundedT)
    codesT = roundedT / halfw_ref[...]
    idxT_f = jnp.sum((roundedT + halfw_ref[...]) * basis_ref[...],
                     axis=0, keepdims=True)
    idxT_ref[...] = idxT_f.astype(jnp.int32)
    zq = lax.dot_general(codesT, fwout_ref[...], (((0,), (0,)), ((), ())),
                         preferred_element_type=f32) + fbout_ref[...][None, :]
    zq_ref[...] = zq

    clsT_ref[...] = lax.dot_general(clsw_ref[...], zq, (((1,), (1,)), ((), ())),
                                    preferred_element_type=f32) + clsb_ref[...][:, None]
    iscT_ref[...] = lax.dot_general(iscw_ref[...], zq, (((1,), (1,)), ((), ())),
                                    preferred_element_type=f32) + iscb_ref[...][:, None]

    d1T = d1_ref[...]
    hd = jnp.maximum(
        lax.dot_general(zq, d1T[:128, :], (((1,), (0,)), ((), ())),
                        preferred_element_type=f32)
        + lax.dot_general(embT, d1T[128:, :], (((0,), (0,)), ((), ())),
                          preferred_element_type=f32)
        + db1_ref[...][None, :], 0.0)
    hd = jnp.maximum(dg(hd, d2_ref[...]) + db2_ref[...][None, :], 0.0)
    pd = dg(hd, d3_ref[...]) + db3_ref[...][None, :]

    d5T = lax.dot_general(wdr_ref[...].reshape(_NT * 12, 32), pd,
                          (((1,), (1,)), ((), ())),
                          preferred_element_type=f32)
    reconT = onehotT[0:1, :] * d5T[0:12, :]
    for t in range(1, _NT):
        reconT = reconT + onehotT[t:t + 1, :] * d5T[12 * t:12 * (t + 1), :]
    reconT_ref[...] = reconT + lax.dot_general(
        bdr_ref[...], onehotT, (((0,), (0,)), ((), ())),
        preferred_element_type=f32)


def _full(shape):
    nd = len(shape)
    return pl.BlockSpec(shape, lambda i: (0,) * nd)


def _rows(width):
    return pl.BlockSpec((_R, width), lambda i: (i, 0))


@jax.jit
def _run(strow, paramsT, args):
    maskf = _sc_route_mask(strow.reshape(_B))
    grid = _B // _R
    in_specs = ([pl.BlockSpec((1, _R), lambda i: (0, i)),
                 pl.BlockSpec((12, _R), lambda i: (0, i))]
                + [_full(a.shape) for a in args])
    out_shapes = (
        jax.ShapeDtypeStruct((12, _B), jnp.float32),
        jax.ShapeDtypeStruct((_NT, _B), jnp.float32),
        jax.ShapeDtypeStruct((2, _B), jnp.float32),
        jax.ShapeDtypeStruct((_B, 128), jnp.float32),
        jax.ShapeDtypeStruct((1, _B), jnp.int32),
    )
    out_specs = (pl.BlockSpec((12, _R), lambda i: (0, i)),
                 pl.BlockSpec((_NT, _R), lambda i: (0, i)),
                 pl.BlockSpec((2, _R), lambda i: (0, i)),
                 pl.BlockSpec((_R, 128), lambda i: (i, 0)),
                 pl.BlockSpec((1, _R), lambda i: (0, i)))
    outs = pl.pallas_call(
        _tc_body,
        grid=(grid,),
        in_specs=in_specs,
        out_specs=out_specs,
        out_shape=out_shapes,
        compiler_params=pltpu.CompilerParams(
            dimension_semantics=("arbitrary",),
        ),
    )(strow, paramsT, *args)
    return outs + (maskf,)


def kernel(params, surface_type, type_emb, W_pe, b_pe,
           enc_W1, enc_b1, enc_W2, enc_b2, enc_W3, enc_b3, enc_W4, enc_b4,
           fsq_Win, fsq_bin, fsq_Wout, fsq_bout,
           dec_W1, dec_b1, dec_W2, dec_b2, dec_W3, dec_b3,
           cls_W, cls_b, isc_W, isc_b, decraw_W, decraw_b):
    strow = surface_type.astype(jnp.int32).reshape(1, _B)
    args = (
        W_pe.transpose(0, 2, 1),
        b_pe,
        enc_W1.T, enc_b1, enc_W2, enc_b2, enc_W3, enc_b3, enc_W4, enc_b4,
        fsq_Win, fsq_bin, fsq_Wout.T, fsq_bout,
        cls_W, cls_b, isc_W, isc_b,
        dec_W1.T, dec_b1, dec_W2, dec_b2, dec_W3, dec_b3,
        decraw_W,
        decraw_b,
        type_emb,
        jnp.asarray(_SHIFT).reshape(4, 1), jnp.asarray(_HALF_L).reshape(4, 1),
        jnp.asarray(_OFFSET).reshape(4, 1), jnp.asarray(_HALF_W).reshape(4, 1),
        jnp.asarray(_BASIS).reshape(4, 1),
    )
    reconT, clsT, iscT, zq, idxT, maskf = _run(strow, params.T, args)
    mask = maskf.T > 0.5
    return reconT.T, mask, clsT.T, iscT.T, zq, idxT.reshape(_B)
